# Initial kernel scaffold; baseline (speedup 1.0000x reference)
import functools

import jax
import jax.numpy as jnp
from jax import lax
from jax.experimental import pallas as pl
from jax.experimental.pallas import tpu as pltpu
from jax.experimental.pallas import tpu_sc as plsc

NU = 6000
NI = 4000
NN = NU + NI
DD = 128
EE = 320000

NC = 2
NS = 16
NW = NC * NS
LL = 16

EPW = EE // NW
CHUNK = 80
NCH = EPW // CHUNK
NGR = CHUNK // LL

RPT = NN // NS
ZR = 125

_SEG = DD // LL


def _splat(v16, j):
    idx = jnp.full((LL,), j, jnp.int32)
    return lax.gather(
        v16, idx[:, None],
        lax.GatherDimensionNumbers(offset_dims=(), collapsed_slice_dims=(0,),
                                   start_index_map=(0,)),
        (1,), mode=lax.GatherScatterMode.PROMISE_IN_BOUNDS)


def _spmm_body(x_hbm, rows_hbm, cols_hbm, vals_hbm, out_hbm,
               rows_v, cols_v, vals_v, gb0, gb1, zbuf, acc_sh, gs0, gs1):
    cid = lax.axis_index("c")
    sid = lax.axis_index("s")
    wid = sid * NC + cid

    zv = jnp.zeros((LL,), jnp.float32)

    def zrow(r, _):
        for j in range(_SEG):
            zbuf[r, pl.ds(j * LL, LL)] = zv
        return 0

    lax.fori_loop(0, ZR, zrow, 0)
    for k in range(RPT // ZR):
        pltpu.sync_copy(zbuf, acc_sh.at[pl.ds(sid * RPT + k * ZR, ZR)])

    pltpu.sync_copy(rows_hbm.at[wid], rows_v)
    pltpu.sync_copy(cols_hbm.at[wid], cols_v)
    pltpu.sync_copy(vals_hbm.at[wid], vals_v)

    plsc.subcore_barrier()

    gbufs = (gb0, gb1)
    gsems = (gs0, gs1)

    def start_gather(c, b):
        pltpu.async_copy(x_hbm.at[cols_v.at[c]], gbufs[b], gsems[b])

    def wait_gather(c, b):
        pltpu.make_async_copy(x_hbm.at[cols_v.at[c]], gbufs[b], gsems[b]).wait()

    def process(c, b):
        gb = gbufs[b]

        def group(g, _):
            v16 = vals_v[pl.ds(c * CHUNK + g * LL, LL)]
            for j in range(LL):
                vs = _splat(v16, j)
                e = g * LL + j
                for s in range(_SEG):
                    gb[e, pl.ds(s * LL, LL)] = gb[e, pl.ds(s * LL, LL)] * vs
            return 0

        lax.fori_loop(0, NGR, group, 0)
        pltpu.sync_copy(gb, acc_sh.at[rows_v.at[c]], add=True)

    start_gather(0, 0)
    start_gather(1, 1)

    def step(it, _):
        for b in range(2):
            c = it * 2 + b
            wait_gather(c, b)
            process(c, b)

            @pl.when(c + 2 < NCH)
            def _():
                start_gather(c + 2, b)
        return 0

    lax.fori_loop(0, (NCH - 1) // 2, step, 0)
    wait_gather(NCH - 1, 0)
    process(NCH - 1, 0)

    plsc.subcore_barrier()
    pltpu.sync_copy(acc_sh.at[pl.ds(sid * RPT, RPT)],
                    out_hbm.at[cid, pl.ds(sid * RPT, RPT)])


@jax.jit
def _spmm(x, rows, cols, vals):
    mesh = plsc.VectorSubcoreMesh(core_axis_name="c", subcore_axis_name="s")
    return pl.kernel(
        _spmm_body,
        out_type=jax.ShapeDtypeStruct((NC, NN, DD), jnp.float32),
        mesh=mesh,
        scratch_types=[
            pltpu.VMEM((NCH, CHUNK), jnp.int32),
            pltpu.VMEM((NCH, CHUNK), jnp.int32),
            pltpu.VMEM((EPW,), jnp.float32),
            pltpu.VMEM((CHUNK, DD), jnp.float32),
            pltpu.VMEM((CHUNK, DD), jnp.float32),
            pltpu.VMEM((ZR, DD), jnp.float32),
            pltpu.VMEM_SHARED((NN, DD), jnp.float32),
            pltpu.SemaphoreType.DMA,
            pltpu.SemaphoreType.DMA,
        ],
    )(x, rows, cols, vals)


BLK = 1000
NBLK = NN // BLK
UB = NU // BLK


def _seg(i):
    return (i >= UB).astype(jnp.int32)


def _proj_body(emb_ref, w_ref, out_ref):
    out_ref[...] = jnp.dot(emb_ref[...], w_ref[0],
                           preferred_element_type=jnp.float32)


@jax.jit
def _proj(emb, w_both):
    return pl.pallas_call(
        _proj_body,
        grid=(NBLK,),
        in_specs=[
            pl.BlockSpec((BLK, DD), lambda i: (i, 0)),
            pl.BlockSpec((1, DD, DD), lambda i: (_seg(i), 0, 0)),
        ],
        out_specs=pl.BlockSpec((BLK, DD), lambda i: (i, 0)),
        out_shape=jax.ShapeDtypeStruct((NN, DD), jnp.float32),
    )(emb, w_both)


def _xtx_body(hyper_ref, x_ref, t_ref):
    i = pl.program_id(0)
    contrib = lax.dot_general(hyper_ref[...], x_ref[...],
                              (((0,), (0,)), ((), ())),
                              preferred_element_type=jnp.float32)

    @pl.when((i == 0) | (i == UB))
    def _():
        t_ref[0] = contrib

    @pl.when((i != 0) & (i != UB))
    def _():
        t_ref[0] += contrib


@jax.jit
def _seg_xtx(hyper, x):
    return pl.pallas_call(
        _xtx_body,
        grid=(NBLK,),
        in_specs=[
            pl.BlockSpec((BLK, DD), lambda i: (i, 0)),
            pl.BlockSpec((BLK, DD), lambda i: (i, 0)),
        ],
        out_specs=pl.BlockSpec((1, DD, DD), lambda i: (_seg(i), 0, 0)),
        out_shape=jax.ShapeDtypeStruct((2, DD, DD), jnp.float32),
    )(hyper, x)


def _fuse_body(hyper_ref, t_ref, p_ref, prev_ref,
               gcn_ref, hgnn_ref, hidden_ref, ssum_ref):
    g = p_ref[0] + p_ref[1]
    h = jnp.dot(hyper_ref[...], t_ref[0], preferred_element_type=jnp.float32)
    gcn_ref[...] = g
    hgnn_ref[...] = h
    hid = g + h
    hidden_ref[...] = hid
    ssum_ref[...] = prev_ref[...] + hid


@jax.jit
def _fuse(hyper, t, p, prev):
    return pl.pallas_call(
        _fuse_body,
        grid=(NBLK,),
        in_specs=[
            pl.BlockSpec((BLK, DD), lambda i: (i, 0)),
            pl.BlockSpec((1, DD, DD), lambda i: (_seg(i), 0, 0)),
            pl.BlockSpec((2, BLK, DD), lambda i: (0, i, 0)),
            pl.BlockSpec((BLK, DD), lambda i: (i, 0)),
        ],
        out_specs=[
            pl.BlockSpec((BLK, DD), lambda i: (i, 0)),
            pl.BlockSpec((BLK, DD), lambda i: (i, 0)),
            pl.BlockSpec((BLK, DD), lambda i: (i, 0)),
            pl.BlockSpec((BLK, DD), lambda i: (i, 0)),
        ],
        out_shape=[jax.ShapeDtypeStruct((NN, DD), jnp.float32)] * 4,
    )(hyper, t, p, prev)


def kernel(user_emb, item_emb, user_w, item_w, adj_indices, adj_vals, keep_rate):
    del keep_rate
    emb = jnp.concatenate([user_emb, item_emb], axis=0)
    w_both = jnp.stack([user_w, item_w], axis=0)
    rows = adj_indices[0].astype(jnp.int32).reshape(NW, NCH, CHUNK)
    cols = adj_indices[1].astype(jnp.int32).reshape(NW, NCH, CHUNK)
    vals = adj_vals.astype(jnp.float32).reshape(NW, EPW)

    hyper = _proj(emb, w_both)
    hidden = emb
    ssum = emb
    gcns, hgnns = [], []
    for _ in range(2):
        p = _spmm(hidden, rows, cols, vals)
        t = _seg_xtx(hyper, hidden)
        gcn, hgnn, hidden, ssum = _fuse(hyper, t, p, ssum)
        gcns.append(gcn)
        hgnns.append(hgnn)

    return (ssum[:NU], ssum[NU:],
            jnp.stack(gcns, axis=0), jnp.stack(hgnns, axis=0))


# speedup vs baseline: 3.6706x; 3.6706x over previous
import jax
import jax.numpy as jnp
from jax import lax
from jax.experimental import pallas as pl
from jax.experimental.pallas import tpu as pltpu
from jax.experimental.pallas import tpu_sc as plsc

NU = 6000
NI = 4000
NN = NU + NI
DD = 128
EE = 320000

NC = 2
NS = 16
LL = 16
FH = DD // NC

EPW = EE // NS
CHUNK = 80
NCH = EPW // CHUNK
NGR = CHUNK // LL

RPT = 624
TAIL = NN - NS * RPT
ZR = 208

_SEG = FH // LL


def _splat(v16, j):
    idx = jnp.full((LL,), j, jnp.int32)
    return lax.gather(
        v16, idx[:, None],
        lax.GatherDimensionNumbers(offset_dims=(), collapsed_slice_dims=(0,),
                                   start_index_map=(0,)),
        (1,), mode=lax.GatherScatterMode.PROMISE_IN_BOUNDS)


def _spmm_body(x0_hbm, x1_hbm, rows_hbm, cols_hbm, vals_hbm, out_hbm,
               rows_v, cols_v, vals_v, gb0, gb1, zbuf, acc_sh, gs0, gs1):
    cid = lax.axis_index("c")
    sid = lax.axis_index("s")

    zv = jnp.zeros((LL,), jnp.float32)

    def zrow(r, _):
        for j in range(_SEG):
            zbuf[r, pl.ds(j * LL, LL)] = zv
        return 0

    lax.fori_loop(0, ZR, zrow, 0)
    for k in range(RPT // ZR):
        pltpu.sync_copy(zbuf, acc_sh.at[pl.ds(sid * RPT + k * ZR, ZR)])

    @pl.when(sid == NS - 1)
    def _():
        pltpu.sync_copy(zbuf.at[pl.ds(0, TAIL)],
                        acc_sh.at[pl.ds(NS * RPT, TAIL)])

    pltpu.sync_copy(rows_hbm.at[sid], rows_v)
    pltpu.sync_copy(cols_hbm.at[sid], cols_v)
    pltpu.sync_copy(vals_hbm.at[sid], vals_v)

    plsc.subcore_barrier()

    gbufs = (gb0, gb1)
    gsems = (gs0, gs1)

    def start_gather(c, b):
        @pl.when(cid == 0)
        def _():
            pltpu.async_copy(x0_hbm.at[cols_v.at[c]], gbufs[b], gsems[b])

        @pl.when(cid == 1)
        def _():
            pltpu.async_copy(x1_hbm.at[cols_v.at[c]], gbufs[b], gsems[b])

    def wait_gather(c, b):
        pltpu.make_async_copy(x0_hbm.at[cols_v.at[c]], gbufs[b],
                              gsems[b]).wait()

    def process(c, b):
        gb = gbufs[b]

        def group(g, _):
            v16 = vals_v[pl.ds(c * CHUNK + g * LL, LL)]
            for j in range(LL):
                vs = _splat(v16, j)
                e = g * LL + j
                for s in range(_SEG):
                    gb[e, pl.ds(s * LL, LL)] = gb[e, pl.ds(s * LL, LL)] * vs
            return 0

        lax.fori_loop(0, NGR, group, 0)
        pltpu.sync_copy(gb, acc_sh.at[rows_v.at[c]], add=True)

    start_gather(0, 0)
    start_gather(1, 1)

    def step(it, _):
        for b in range(2):
            c = it * 2 + b
            wait_gather(c, b)
            process(c, b)

            @pl.when(c + 2 < NCH)
            def _():
                start_gather(c + 2, b)
        return 0

    lax.fori_loop(0, NCH // 2, step, 0)

    plsc.subcore_barrier()
    pltpu.sync_copy(acc_sh.at[pl.ds(sid * RPT, RPT)],
                    out_hbm.at[cid, pl.ds(sid * RPT, RPT)])

    @pl.when(sid == NS - 1)
    def _():
        pltpu.sync_copy(acc_sh.at[pl.ds(NS * RPT, TAIL)],
                        out_hbm.at[cid, pl.ds(NS * RPT, TAIL)])


@jax.jit
def _spmm(x0, x1, rows, cols, vals):
    mesh = plsc.VectorSubcoreMesh(core_axis_name="c", subcore_axis_name="s")
    return pl.kernel(
        _spmm_body,
        out_type=jax.ShapeDtypeStruct((NC, NN, FH), jnp.float32),
        mesh=mesh,
        scratch_types=[
            pltpu.VMEM((NCH, CHUNK), jnp.int32),
            pltpu.VMEM((NCH, CHUNK), jnp.int32),
            pltpu.VMEM((EPW,), jnp.float32),
            pltpu.VMEM((CHUNK, FH), jnp.float32),
            pltpu.VMEM((CHUNK, FH), jnp.float32),
            pltpu.VMEM((ZR, FH), jnp.float32),
            pltpu.VMEM_SHARED((NN, FH), jnp.float32),
            pltpu.SemaphoreType.DMA,
            pltpu.SemaphoreType.DMA,
        ],
        compiler_params=pltpu.CompilerParams(use_tc_tiling_on_sc=False),
    )(x0, x1, rows, cols, vals)


BLK = 1000
NBLK = NN // BLK
UB = NU // BLK


def _seg(i):
    return (i >= UB).astype(jnp.int32)


def _proj_body(emb_ref, w_ref, out_ref):
    out_ref[...] = jnp.dot(emb_ref[...], w_ref[0],
                           preferred_element_type=jnp.float32)


@jax.jit
def _proj(emb, w_both):
    return pl.pallas_call(
        _proj_body,
        grid=(NBLK,),
        in_specs=[
            pl.BlockSpec((BLK, DD), lambda i: (i, 0)),
            pl.BlockSpec((1, DD, DD), lambda i: (_seg(i), 0, 0)),
        ],
        out_specs=pl.BlockSpec((BLK, DD), lambda i: (i, 0)),
        out_shape=jax.ShapeDtypeStruct((NN, DD), jnp.float32),
    )(emb, w_both)


def _xtx_body(hyper_ref, x_ref, t_ref):
    i = pl.program_id(0)
    contrib = lax.dot_general(hyper_ref[...], x_ref[...],
                              (((0,), (0,)), ((), ())),
                              preferred_element_type=jnp.float32)

    @pl.when((i == 0) | (i == UB))
    def _():
        t_ref[0] = contrib

    @pl.when((i != 0) & (i != UB))
    def _():
        t_ref[0] += contrib


@jax.jit
def _seg_xtx(hyper, x):
    return pl.pallas_call(
        _xtx_body,
        grid=(NBLK,),
        in_specs=[
            pl.BlockSpec((BLK, DD), lambda i: (i, 0)),
            pl.BlockSpec((BLK, DD), lambda i: (i, 0)),
        ],
        out_specs=pl.BlockSpec((1, DD, DD), lambda i: (_seg(i), 0, 0)),
        out_shape=jax.ShapeDtypeStruct((2, DD, DD), jnp.float32),
    )(hyper, x)


def _fuse_body(hyper_ref, t_ref, p_ref, prev_ref,
               gcn_ref, hgnn_ref, hidden_ref, ssum_ref):
    g = jnp.concatenate([p_ref[0], p_ref[1]], axis=-1)
    h = jnp.dot(hyper_ref[...], t_ref[0], preferred_element_type=jnp.float32)
    gcn_ref[...] = g
    hgnn_ref[...] = h
    hid = g + h
    hidden_ref[...] = hid
    ssum_ref[...] = prev_ref[...] + hid


@jax.jit
def _fuse(hyper, t, p, prev):
    return pl.pallas_call(
        _fuse_body,
        grid=(NBLK,),
        in_specs=[
            pl.BlockSpec((BLK, DD), lambda i: (i, 0)),
            pl.BlockSpec((1, DD, DD), lambda i: (_seg(i), 0, 0)),
            pl.BlockSpec((2, BLK, FH), lambda i: (0, i, 0)),
            pl.BlockSpec((BLK, DD), lambda i: (i, 0)),
        ],
        out_specs=[
            pl.BlockSpec((BLK, DD), lambda i: (i, 0)),
            pl.BlockSpec((BLK, DD), lambda i: (i, 0)),
            pl.BlockSpec((BLK, DD), lambda i: (i, 0)),
            pl.BlockSpec((BLK, DD), lambda i: (i, 0)),
        ],
        out_shape=[jax.ShapeDtypeStruct((NN, DD), jnp.float32)] * 4,
    )(hyper, t, p, prev)


def kernel(user_emb, item_emb, user_w, item_w, adj_indices, adj_vals, keep_rate):
    del keep_rate
    emb = jnp.concatenate([user_emb, item_emb], axis=0)
    w_both = jnp.stack([user_w, item_w], axis=0)
    rows = adj_indices[0].astype(jnp.int32).reshape(NS, NCH, CHUNK)
    cols = adj_indices[1].astype(jnp.int32).reshape(NS, NCH, CHUNK)
    vals = adj_vals.astype(jnp.float32).reshape(NS, EPW)

    hyper = _proj(emb, w_both)
    hidden = emb
    ssum = emb
    gcns, hgnns = [], []
    for _ in range(2):
        p = _spmm(hidden[:, :FH], hidden[:, FH:], rows, cols, vals)
        t = _seg_xtx(hyper, hidden)
        gcn, hgnn, hidden, ssum = _fuse(hyper, t, p, ssum)
        gcns.append(gcn)
        hgnns.append(hgnn)

    return (ssum[:NU], ssum[NU:],
            jnp.stack(gcns, axis=0), jnp.stack(hgnns, axis=0))


# speedup vs baseline: 7.9508x; 2.1661x over previous
import jax
import jax.numpy as jnp
from jax import lax
from jax.experimental import pallas as pl
from jax.experimental.pallas import tpu as pltpu
from jax.experimental.pallas import tpu_sc as plsc

NU = 6000
NI = 4000
NN = NU + NI
DD = 128
EE = 320000

NC = 2
NS = 16
LL = 16
FH = DD // NC

EPW = EE // NS
CHUNK = 80
NCH = EPW // CHUNK
NGR = CHUNK // LL

RPT = 624
TAIL = NN - NS * RPT
ZR = 80

_SEG = FH // LL


def _splat(v16, j):
    idx = jnp.full((LL,), j, jnp.int32)
    return lax.gather(
        v16, idx[:, None],
        lax.GatherDimensionNumbers(offset_dims=(), collapsed_slice_dims=(0,),
                                   start_index_map=(0,)),
        (1,), mode=lax.GatherScatterMode.PROMISE_IN_BOUNDS)


def _spmm_body(x0_hbm, x1_hbm, rows_hbm, cols_hbm, vals_hbm, out_hbm,
               rows_v, cols_v, vals_v, gb0, gb1, sb0, sb1, acc_sh,
               gs0, gs1, ss0, ss1):
    cid = lax.axis_index("c")
    sid = lax.axis_index("s")

    zv = jnp.zeros((LL,), jnp.float32)

    def zrow(r, _):
        for j in range(_SEG):
            gb0[r, pl.ds(j * LL, LL)] = zv
        return 0

    lax.fori_loop(0, ZR, zrow, 0)
    for k in range(RPT // ZR):
        pltpu.sync_copy(gb0, acc_sh.at[pl.ds(sid * RPT + k * ZR, ZR)])
    pltpu.sync_copy(gb0.at[pl.ds(0, RPT - (RPT // ZR) * ZR)],
                    acc_sh.at[pl.ds(sid * RPT + (RPT // ZR) * ZR,
                                    RPT - (RPT // ZR) * ZR)])

    @pl.when(sid == NS - 1)
    def _():
        pltpu.sync_copy(gb0.at[pl.ds(0, TAIL)],
                        acc_sh.at[pl.ds(NS * RPT, TAIL)])

    pltpu.sync_copy(rows_hbm.at[sid], rows_v)
    pltpu.sync_copy(cols_hbm.at[sid], cols_v)
    pltpu.sync_copy(vals_hbm.at[sid], vals_v)

    plsc.subcore_barrier()

    gbufs = (gb0, gb1)
    sbufs = (sb0, sb1)
    gsems = (gs0, gs1)
    ssems = (ss0, ss1)

    def start_gather(c, b):
        @pl.when(cid == 0)
        def _():
            pltpu.async_copy(x0_hbm.at[cols_v.at[c]], gbufs[b], gsems[b])

        @pl.when(cid == 1)
        def _():
            pltpu.async_copy(x1_hbm.at[cols_v.at[c]], gbufs[b], gsems[b])

    def wait_gather(c, b):
        pltpu.make_async_copy(x0_hbm.at[cols_v.at[c]], gbufs[b],
                              gsems[b]).wait()

    def start_scatter(c, b):
        pltpu.async_copy(sbufs[b], acc_sh.at[rows_v.at[c]], ssems[b],
                         add=True)

    def wait_scatter(c, b):
        pltpu.make_async_copy(sbufs[b], acc_sh.at[rows_v.at[c]],
                              ssems[b]).wait()

    def process(c, b):
        gb = gbufs[b]
        sb = sbufs[b]

        def group(g, _):
            v16 = vals_v[pl.ds(c * CHUNK + g * LL, LL)]
            for q in range(LL // 4):
                vs = [_splat(v16, q * 4 + r) for r in range(4)]
                xs = [[gb[g * LL + q * 4 + r, pl.ds(s * LL, LL)]
                       for s in range(_SEG)] for r in range(4)]
                for r in range(4):
                    for s in range(_SEG):
                        sb[g * LL + q * 4 + r, pl.ds(s * LL, LL)] = (
                            xs[r][s] * vs[r])
            return 0

        lax.fori_loop(0, NGR, group, 0)

    start_gather(0, 0)
    start_gather(1, 1)

    def step(it, _):
        for b in range(2):
            c = it * 2 + b
            wait_gather(c, b)

            @pl.when(it >= 1)
            def _():
                wait_scatter(c - 2, b)

            process(c, b)
            start_scatter(c, b)

            @pl.when(c + 2 < NCH)
            def _():
                start_gather(c + 2, b)
        return 0

    lax.fori_loop(0, NCH // 2, step, 0)
    wait_scatter(NCH - 2, 0)
    wait_scatter(NCH - 1, 1)

    plsc.subcore_barrier()
    pltpu.sync_copy(acc_sh.at[pl.ds(sid * RPT, RPT)],
                    out_hbm.at[cid, pl.ds(sid * RPT, RPT)])

    @pl.when(sid == NS - 1)
    def _():
        pltpu.sync_copy(acc_sh.at[pl.ds(NS * RPT, TAIL)],
                        out_hbm.at[cid, pl.ds(NS * RPT, TAIL)])


@jax.jit
def _spmm(x0, x1, rows, cols, vals):
    mesh = plsc.VectorSubcoreMesh(core_axis_name="c", subcore_axis_name="s")
    return pl.kernel(
        _spmm_body,
        out_type=jax.ShapeDtypeStruct((NC, NN, FH), jnp.float32),
        mesh=mesh,
        scratch_types=[
            pltpu.VMEM((NCH, CHUNK), jnp.int32),
            pltpu.VMEM((NCH, CHUNK), jnp.int32),
            pltpu.VMEM((EPW,), jnp.float32),
            pltpu.VMEM((CHUNK, FH), jnp.float32),
            pltpu.VMEM((CHUNK, FH), jnp.float32),
            pltpu.VMEM((CHUNK, FH), jnp.float32),
            pltpu.VMEM((CHUNK, FH), jnp.float32),
            pltpu.VMEM_SHARED((NN, FH), jnp.float32),
            pltpu.SemaphoreType.DMA,
            pltpu.SemaphoreType.DMA,
            pltpu.SemaphoreType.DMA,
            pltpu.SemaphoreType.DMA,
        ],
        compiler_params=pltpu.CompilerParams(use_tc_tiling_on_sc=False),
    )(x0, x1, rows, cols, vals)


BLK = 1000
NBLK = NN // BLK
UB = NU // BLK


def _seg(i):
    return (i >= UB).astype(jnp.int32)


def _proj_body(emb_ref, w_ref, out_ref):
    out_ref[...] = jnp.dot(emb_ref[...], w_ref[0],
                           preferred_element_type=jnp.float32)


@jax.jit
def _proj(emb, w_both):
    return pl.pallas_call(
        _proj_body,
        grid=(NBLK,),
        in_specs=[
            pl.BlockSpec((BLK, DD), lambda i: (i, 0)),
            pl.BlockSpec((1, DD, DD), lambda i: (_seg(i), 0, 0)),
        ],
        out_specs=pl.BlockSpec((BLK, DD), lambda i: (i, 0)),
        out_shape=jax.ShapeDtypeStruct((NN, DD), jnp.float32),
    )(emb, w_both)


def _xtx_body(hyper_ref, x_ref, t_ref):
    i = pl.program_id(0)
    contrib = lax.dot_general(hyper_ref[...], x_ref[...],
                              (((0,), (0,)), ((), ())),
                              preferred_element_type=jnp.float32)

    @pl.when((i == 0) | (i == UB))
    def _():
        t_ref[0] = contrib

    @pl.when((i != 0) & (i != UB))
    def _():
        t_ref[0] += contrib


@jax.jit
def _seg_xtx(hyper, x):
    return pl.pallas_call(
        _xtx_body,
        grid=(NBLK,),
        in_specs=[
            pl.BlockSpec((BLK, DD), lambda i: (i, 0)),
            pl.BlockSpec((BLK, DD), lambda i: (i, 0)),
        ],
        out_specs=pl.BlockSpec((1, DD, DD), lambda i: (_seg(i), 0, 0)),
        out_shape=jax.ShapeDtypeStruct((2, DD, DD), jnp.float32),
    )(hyper, x)


def _fuse_body(hyper_ref, t_ref, p_ref, prev_ref,
               gcn_ref, hgnn_ref, hidden_ref, ssum_ref):
    g = jnp.concatenate([p_ref[0], p_ref[1]], axis=-1)
    h = jnp.dot(hyper_ref[...], t_ref[0], preferred_element_type=jnp.float32)
    gcn_ref[...] = g
    hgnn_ref[...] = h
    hid = g + h
    hidden_ref[...] = hid
    ssum_ref[...] = prev_ref[...] + hid


@jax.jit
def _fuse(hyper, t, p, prev):
    return pl.pallas_call(
        _fuse_body,
        grid=(NBLK,),
        in_specs=[
            pl.BlockSpec((BLK, DD), lambda i: (i, 0)),
            pl.BlockSpec((1, DD, DD), lambda i: (_seg(i), 0, 0)),
            pl.BlockSpec((2, BLK, FH), lambda i: (0, i, 0)),
            pl.BlockSpec((BLK, DD), lambda i: (i, 0)),
        ],
        out_specs=[
            pl.BlockSpec((BLK, DD), lambda i: (i, 0)),
            pl.BlockSpec((BLK, DD), lambda i: (i, 0)),
            pl.BlockSpec((BLK, DD), lambda i: (i, 0)),
            pl.BlockSpec((BLK, DD), lambda i: (i, 0)),
        ],
        out_shape=[jax.ShapeDtypeStruct((NN, DD), jnp.float32)] * 4,
    )(hyper, t, p, prev)


def kernel(user_emb, item_emb, user_w, item_w, adj_indices, adj_vals, keep_rate):
    del keep_rate
    emb = jnp.concatenate([user_emb, item_emb], axis=0)
    w_both = jnp.stack([user_w, item_w], axis=0)
    rows = adj_indices[0].astype(jnp.int32).reshape(NS, NCH, CHUNK)
    cols = adj_indices[1].astype(jnp.int32).reshape(NS, NCH, CHUNK)
    vals = adj_vals.astype(jnp.float32).reshape(NS, EPW)

    hyper = _proj(emb, w_both)
    hidden = emb
    ssum = emb
    gcns, hgnns = [], []
    for _ in range(2):
        p = _spmm(hidden[:, :FH], hidden[:, FH:], rows, cols, vals)
        t = _seg_xtx(hyper, hidden)
        gcn, hgnn, hidden, ssum = _fuse(hyper, t, p, ssum)
        gcns.append(gcn)
        hgnns.append(hgnn)

    return (ssum[:NU], ssum[NU:],
            jnp.stack(gcns, axis=0), jnp.stack(hgnns, axis=0))
